# baseline (speedup 1.0000x reference)
import jax, jax.numpy as jnp
import numpy as np
from jax.experimental import pallas as pl

_STRIDE = 8
_SIZES = (32.0, 64.0, 128.0, 256.0, 512.0)
_PRE_NMS = 2000
_POST_NMS = 1000
_NMS_THRESH = 0.7
_MIN_SIZE = 0.0
_BBOX_CLIP = float(np.log(1000.0 / 16.0))


def _grid_anchors_k(hf, wf):
    sizes = jnp.asarray(_SIZES, dtype=jnp.float32)
    base = jnp.stack([-sizes / 2, -sizes / 2, sizes / 2, sizes / 2], axis=1)
    sx = jnp.arange(wf, dtype=jnp.float32) * _STRIDE
    sy = jnp.arange(hf, dtype=jnp.float32) * _STRIDE
    yy, xx = jnp.meshgrid(sy, sx, indexing='ij')
    shifts = jnp.stack([xx.ravel(), yy.ravel(), xx.ravel(), yy.ravel()], axis=1)
    anchors = (shifts[:, None, :] + base[None, :, :]).reshape(-1, 4)
    return anchors


def _conv_k(x, w, b):
    y = jax.lax.conv_general_dilated(x, w, (1, 1), 'SAME', dimension_numbers=('NCHW', 'OIHW', 'NCHW'))
    return y + b[None, :, None, None]


def _pairwise_iou_k(boxes):
    x1, y1, x2, y2 = boxes[:, 0], boxes[:, 1], boxes[:, 2], boxes[:, 3]
    area = jnp.maximum(x2 - x1, 0.0) * jnp.maximum(y2 - y1, 0.0)
    ix1 = jnp.maximum(x1[:, None], x1[None, :])
    iy1 = jnp.maximum(y1[:, None], y1[None, :])
    ix2 = jnp.minimum(x2[:, None], x2[None, :])
    iy2 = jnp.minimum(y2[:, None], y2[None, :])
    iw = jnp.maximum(ix2 - ix1, 0.0)
    ih = jnp.maximum(iy2 - iy1, 0.0)
    inter = iw * ih
    union = area[:, None] + area[None, :] - inter
    return inter / jnp.maximum(union, 1e-9)


def kernel(features, W_conv, b_conv, W_obj, b_obj, W_reg, b_reg, image_height, image_width):
    t = jax.nn.relu(_conv_k(features, W_conv, b_conv))
    objectness = _conv_k(t, W_obj, b_obj)
    box_reg = _conv_k(t, W_reg, b_reg)
    B, A, Hf, Wf = objectness.shape
    anchors = _grid_anchors_k(Hf, Wf)
    obj = jnp.transpose(objectness, (0, 2, 3, 1)).reshape(-1)
    obj = jax.nn.sigmoid(obj)
    reg = box_reg.reshape(B, A, 4, Hf, Wf)
    reg = jnp.transpose(reg, (0, 3, 4, 1, 2)).reshape(-1, 4)
    scores, idx = jax.lax.top_k(obj, _PRE_NMS)
    deltas = reg[idx]
    anch = anchors[idx]
    widths = anch[:, 2] - anch[:, 0]
    heights = anch[:, 3] - anch[:, 1]
    ctr_x = anch[:, 0] + 0.5 * widths
    ctr_y = anch[:, 1] + 0.5 * heights
    dx, dy, dw, dh = deltas[:, 0], deltas[:, 1], deltas[:, 2], deltas[:, 3]
    dw = jnp.minimum(dw, _BBOX_CLIP)
    dh = jnp.minimum(dh, _BBOX_CLIP)
    pred_ctr_x = dx * widths + ctr_x
    pred_ctr_y = dy * heights + ctr_y
    pred_w = jnp.exp(dw) * widths
    pred_h = jnp.exp(dh) * heights
    x1 = pred_ctr_x - 0.5 * pred_w
    y1 = pred_ctr_y - 0.5 * pred_h
    x2 = pred_ctr_x + 0.5 * pred_w
    y2 = pred_ctr_y + 0.5 * pred_h
    W_img = jnp.float32(image_width)
    H_img = jnp.float32(image_height)
    x1 = jnp.clip(x1, 0.0, W_img - 1.0)
    y1 = jnp.clip(y1, 0.0, H_img - 1.0)
    x2 = jnp.clip(x2, 0.0, W_img - 1.0)
    y2 = jnp.clip(y2, 0.0, H_img - 1.0)
    boxes = jnp.stack([x1, y1, x2, y2], axis=1)
    keep = ((x2 - x1) >= _MIN_SIZE) & ((y2 - y1) >= _MIN_SIZE)
    scores = jnp.where(keep, scores, -1.0)
    order = jnp.argsort(-scores)
    boxes_s = boxes[order]
    scores_s = scores[order]
    iou = _pairwise_iou_k(boxes_s)
    K = boxes_s.shape[0]
    rng = jnp.arange(K)

    def body(i, supp):
        alive = jnp.logical_not(supp[i])
        over = iou[i] > _NMS_THRESH
        return supp | (alive & over & (rng > i))

    supp = jax.lax.fori_loop(0, K, body, jnp.zeros((K,), dtype=bool))
    final_scores_all = jnp.where(supp, -1e9, scores_s)
    top_scores, top_idx = jax.lax.top_k(final_scores_all, _POST_NMS)
    final_boxes = boxes_s[top_idx]
    return final_boxes, top_scores


# speedup vs baseline: 5.8156x; 5.8156x over previous
import jax
import jax.numpy as jnp
import numpy as np
from jax.experimental import pallas as pl
from jax.experimental.pallas import tpu as pltpu

_STRIDE = 8
_PRE_NMS = 2000
_POST_NMS = 1000
_NMS_THRESH = 0.7
_BBOX_CLIP = float(np.log(1000.0 / 16.0))

_K = 2048
_T = 128
_NT = _K // _T


def _conv(x, w, b):
    y = jax.lax.conv_general_dilated(x, w, (1, 1), 'SAME',
                                     dimension_numbers=('NCHW', 'OIHW', 'NCHW'))
    return y + b[None, :, None, None]


def _decode(idxf, dx, dy, dw, dh, wlim, hlim, wf):
    spatial = jnp.floor(idxf / 5.0)
    af = idxf - 5.0 * spatial
    hrow = jnp.floor(spatial / wf)
    wcol = spatial - wf * hrow
    size = jnp.where(af == 0.0, 32.0,
           jnp.where(af == 1.0, 64.0,
           jnp.where(af == 2.0, 128.0,
           jnp.where(af == 3.0, 256.0, 512.0))))
    cx = wcol * 8.0
    cy = hrow * 8.0
    dw = jnp.minimum(dw, _BBOX_CLIP)
    dh = jnp.minimum(dh, _BBOX_CLIP)
    pcx = dx * size + cx
    pcy = dy * size + cy
    pw = jnp.exp(dw) * size
    ph = jnp.exp(dh) * size
    x1 = pcx - 0.5 * pw
    y1 = pcy - 0.5 * ph
    x2 = pcx + 0.5 * pw
    y2 = pcy + 0.5 * ph
    x1 = jnp.clip(x1, 0.0, wlim)
    y1 = jnp.clip(y1, 0.0, hlim)
    x2 = jnp.clip(x2, 0.0, wlim)
    y2 = jnp.clip(y2, 0.0, hlim)
    return x1, y1, x2, y2


def _nms_kernel(idx_a_ref, del_a_ref, idx_b_ref, del_b_ref, lim_ref,
                supp_ref, boxes_ref, over_s):
    wlim = lim_ref[0, 0]
    hlim = lim_ref[1, 0]
    wf = lim_ref[2, 0]

    idxa = idx_a_ref[0:1, :].astype(jnp.float32)
    ax1, ay1, ax2, ay2 = _decode(idxa,
                                 del_a_ref[0:1, :], del_a_ref[1:2, :],
                                 del_a_ref[2:3, :], del_a_ref[3:4, :],
                                 wlim, hlim, wf)
    area_a = jnp.maximum(ax2 - ax1, 0.0) * jnp.maximum(ay2 - ay1, 0.0)

    idxb = idx_b_ref[:, 0:1].astype(jnp.float32)
    bx1, by1, bx2, by2 = _decode(idxb,
                                 del_b_ref[:, 0:1], del_b_ref[:, 1:2],
                                 del_b_ref[:, 2:3], del_b_ref[:, 3:4],
                                 wlim, hlim, wf)
    area_b = jnp.maximum(bx2 - bx1, 0.0) * jnp.maximum(by2 - by1, 0.0)

    col = jax.lax.broadcasted_iota(jnp.int32, (1, _K), 1)
    supp = jnp.zeros((1, _K), jnp.float32)

    for ti in range(_NT):
        r0 = ti * _T
        tx1 = bx1[r0:r0 + _T, :]
        ty1 = by1[r0:r0 + _T, :]
        tx2 = bx2[r0:r0 + _T, :]
        ty2 = by2[r0:r0 + _T, :]
        tarea = area_b[r0:r0 + _T, :]
        ix1 = jnp.maximum(tx1, ax1)
        iy1 = jnp.maximum(ty1, ay1)
        ix2 = jnp.minimum(tx2, ax2)
        iy2 = jnp.minimum(ty2, ay2)
        iw = jnp.maximum(ix2 - ix1, 0.0)
        ih = jnp.maximum(iy2 - iy1, 0.0)
        inter = iw * ih
        union = tarea + area_a - inter
        iou = inter / jnp.maximum(union, 1e-9)
        over_s[...] = (iou > _NMS_THRESH).astype(jnp.float32)

        def body(r, supp):
            g = r0 + r
            alive = 1.0 - jnp.sum(supp * (col == g).astype(jnp.float32))
            row = over_s[pl.ds(r, 1), :]
            upd = alive * row * (col > g).astype(jnp.float32)
            return jnp.maximum(supp, upd)

        supp = jax.lax.fori_loop(0, _T, body, supp)

    supp_ref[...] = jnp.broadcast_to(supp, (8, _K))
    boxes_ref[...] = jnp.zeros((_K, 8), jnp.float32)
    boxes_ref[:, 0:1] = bx1
    boxes_ref[:, 1:2] = by1
    boxes_ref[:, 2:3] = bx2
    boxes_ref[:, 3:4] = by2


def _run_nms(idx, deltas, wlim, hlim, wf):
    pad = _K - _PRE_NMS
    idx_p = jnp.pad(idx.astype(jnp.int32), (0, pad))
    del_p = jnp.pad(deltas, ((0, pad), (0, 0)))
    idx_a = jnp.broadcast_to(idx_p[None, :], (8, _K))
    idx_b = jnp.broadcast_to(idx_p[:, None], (_K, 8))
    del_a = jnp.pad(del_p.T, ((0, 4), (0, 0)))
    del_b = jnp.pad(del_p, ((0, 0), (0, 4)))
    lim = jnp.broadcast_to(
        jnp.stack([wlim, hlim, wf])[:, None], (3, 128))
    lim = jnp.pad(lim, ((0, 5), (0, 0)))
    supp, boxes = pl.pallas_call(
        _nms_kernel,
        out_shape=(jax.ShapeDtypeStruct((8, _K), jnp.float32),
                   jax.ShapeDtypeStruct((_K, 8), jnp.float32)),
        scratch_shapes=[pltpu.VMEM((_T, _K), jnp.float32)],
    )(idx_a, del_a, idx_b, del_b, lim)
    return supp[0, :_PRE_NMS] > 0.0, boxes[:_PRE_NMS, 0:4]


def kernel(features, W_conv, b_conv, W_obj, b_obj, W_reg, b_reg,
           image_height, image_width):
    t = jax.nn.relu(_conv(features, W_conv, b_conv))
    objectness = _conv(t, W_obj, b_obj)
    box_reg = _conv(t, W_reg, b_reg)
    B, A, Hf, Wf = objectness.shape
    obj = jnp.transpose(objectness, (0, 2, 3, 1)).reshape(-1)
    obj = jax.nn.sigmoid(obj)
    reg = box_reg.reshape(B, A, 4, Hf, Wf)
    reg = jnp.transpose(reg, (0, 3, 4, 1, 2)).reshape(-1, 4)
    scores, idx = jax.lax.top_k(obj, _PRE_NMS)
    deltas = reg[idx]

    wlim = jnp.float32(image_width) - 1.0
    hlim = jnp.float32(image_height) - 1.0
    supp, boxes_s = _run_nms(idx, deltas, wlim, hlim, jnp.float32(Wf))

    final_scores_all = jnp.where(supp, -1e9, scores)
    top_scores, top_idx = jax.lax.top_k(final_scores_all, _POST_NMS)
    final_boxes = boxes_s[top_idx]
    return final_boxes, top_scores


# speedup vs baseline: 6.2405x; 1.0731x over previous
import jax
import jax.numpy as jnp
import numpy as np
from jax.experimental import pallas as pl
from jax.experimental.pallas import tpu as pltpu

_STRIDE = 8
_PRE_NMS = 2000
_POST_NMS = 1000
_NMS_THRESH = 0.7
_BBOX_CLIP = float(np.log(1000.0 / 16.0))

_K = 2048
_T = 128
_NT = _K // _T


def _conv(x, w, b):
    y = jax.lax.conv_general_dilated(x, w, (1, 1), 'SAME',
                                     dimension_numbers=('NCHW', 'OIHW', 'NCHW'))
    return y + b[None, :, None, None]


def _decode(idxf, dx, dy, dw, dh, wlim, hlim, wf):
    spatial = jnp.floor(idxf / 5.0)
    af = idxf - 5.0 * spatial
    hrow = jnp.floor(spatial / wf)
    wcol = spatial - wf * hrow
    size = jnp.where(af == 0.0, 32.0,
           jnp.where(af == 1.0, 64.0,
           jnp.where(af == 2.0, 128.0,
           jnp.where(af == 3.0, 256.0, 512.0))))
    cx = wcol * 8.0
    cy = hrow * 8.0
    dw = jnp.minimum(dw, _BBOX_CLIP)
    dh = jnp.minimum(dh, _BBOX_CLIP)
    pcx = dx * size + cx
    pcy = dy * size + cy
    pw = jnp.exp(dw) * size
    ph = jnp.exp(dh) * size
    x1 = pcx - 0.5 * pw
    y1 = pcy - 0.5 * ph
    x2 = pcx + 0.5 * pw
    y2 = pcy + 0.5 * ph
    x1 = jnp.clip(x1, 0.0, wlim)
    y1 = jnp.clip(y1, 0.0, hlim)
    x2 = jnp.clip(x2, 0.0, wlim)
    y2 = jnp.clip(y2, 0.0, hlim)
    return x1, y1, x2, y2


def _nms_kernel(idx_a_ref, del_a_ref, idx_b_ref, del_b_ref, lim_ref,
                supp_ref, boxes_ref, over_s, supp_s):
    wlim = lim_ref[0, 0]
    hlim = lim_ref[1, 0]
    wf = lim_ref[2, 0]

    idxa = idx_a_ref[0:1, :].astype(jnp.float32)
    ax1, ay1, ax2, ay2 = _decode(idxa,
                                 del_a_ref[0:1, :], del_a_ref[1:2, :],
                                 del_a_ref[2:3, :], del_a_ref[3:4, :],
                                 wlim, hlim, wf)
    area_a = jnp.maximum(ax2 - ax1, 0.0) * jnp.maximum(ay2 - ay1, 0.0)

    idxb = idx_b_ref[:, 0:1].astype(jnp.float32)
    bx1, by1, bx2, by2 = _decode(idxb,
                                 del_b_ref[:, 0:1], del_b_ref[:, 1:2],
                                 del_b_ref[:, 2:3], del_b_ref[:, 3:4],
                                 wlim, hlim, wf)
    area_b = jnp.maximum(bx2 - bx1, 0.0) * jnp.maximum(by2 - by1, 0.0)

    tl_col = jax.lax.broadcasted_iota(jnp.int32, (1, _T), 1)
    supp_s[...] = jnp.zeros((1, _K), jnp.float32)

    for ti in range(_NT):
        r0 = ti * _T
        tx1 = bx1[r0:r0 + _T, :]
        ty1 = by1[r0:r0 + _T, :]
        tx2 = bx2[r0:r0 + _T, :]
        ty2 = by2[r0:r0 + _T, :]
        tarea = area_b[r0:r0 + _T, :]
        ix1 = jnp.maximum(tx1, ax1[:, r0:])
        iy1 = jnp.maximum(ty1, ay1[:, r0:])
        ix2 = jnp.minimum(tx2, ax2[:, r0:])
        iy2 = jnp.minimum(ty2, ay2[:, r0:])
        iw = jnp.maximum(ix2 - ix1, 0.0)
        ih = jnp.maximum(iy2 - iy1, 0.0)
        inter = iw * ih
        union = tarea + area_a[:, r0:] - inter
        iou = inter / jnp.maximum(union, 1e-9)
        overb = (iou > _NMS_THRESH).astype(jnp.float32)
        over_s[...] = overb[:, 0:_T]

        def body(r, m):
            alive = 1.0 - jnp.sum(m * (tl_col == r).astype(jnp.float32))
            row = over_s[pl.ds(r, 1), :]
            return jnp.maximum(m, alive * row * (tl_col > r).astype(jnp.float32))

        m = jax.lax.fori_loop(0, _T, body, supp_s[0:1, r0:r0 + _T])
        supp_s[0:1, r0:r0 + _T] = m

        if r0 + _T < _K:
            alive_vec = 1.0 - m
            contrib = jnp.dot(alive_vec, overb[:, _T:],
                              preferred_element_type=jnp.float32)
            supp_s[0:1, r0 + _T:] = jnp.maximum(
                supp_s[0:1, r0 + _T:], (contrib > 0.5).astype(jnp.float32))

    supp_ref[...] = jnp.broadcast_to(supp_s[0:1, :], (8, _K))
    boxes_ref[...] = jnp.zeros((_K, 8), jnp.float32)
    boxes_ref[:, 0:1] = bx1
    boxes_ref[:, 1:2] = by1
    boxes_ref[:, 2:3] = bx2
    boxes_ref[:, 3:4] = by2


def _run_nms(idx, deltas, wlim, hlim, wf):
    pad = _K - _PRE_NMS
    idx_p = jnp.pad(idx.astype(jnp.int32), (0, pad))
    del_p = jnp.pad(deltas, ((0, pad), (0, 0)))
    idx_a = jnp.broadcast_to(idx_p[None, :], (8, _K))
    idx_b = jnp.broadcast_to(idx_p[:, None], (_K, 8))
    del_a = jnp.pad(del_p.T, ((0, 4), (0, 0)))
    del_b = jnp.pad(del_p, ((0, 0), (0, 4)))
    lim = jnp.broadcast_to(
        jnp.stack([wlim, hlim, wf])[:, None], (3, 128))
    lim = jnp.pad(lim, ((0, 5), (0, 0)))
    supp, boxes = pl.pallas_call(
        _nms_kernel,
        out_shape=(jax.ShapeDtypeStruct((8, _K), jnp.float32),
                   jax.ShapeDtypeStruct((_K, 8), jnp.float32)),
        scratch_shapes=[pltpu.VMEM((_T, _T), jnp.float32),
                        pltpu.VMEM((1, _K), jnp.float32)],
    )(idx_a, del_a, idx_b, del_b, lim)
    return supp[0, :_PRE_NMS] > 0.0, boxes[:_PRE_NMS, 0:4]


def kernel(features, W_conv, b_conv, W_obj, b_obj, W_reg, b_reg,
           image_height, image_width):
    t = jax.nn.relu(_conv(features, W_conv, b_conv))
    objectness = _conv(t, W_obj, b_obj)
    box_reg = _conv(t, W_reg, b_reg)
    B, A, Hf, Wf = objectness.shape
    obj = jnp.transpose(objectness, (0, 2, 3, 1)).reshape(-1)
    obj = jax.nn.sigmoid(obj)
    reg = box_reg.reshape(B, A, 4, Hf, Wf)
    reg = jnp.transpose(reg, (0, 3, 4, 1, 2)).reshape(-1, 4)
    scores, idx = jax.lax.top_k(obj, _PRE_NMS)
    deltas = reg[idx]

    wlim = jnp.float32(image_width) - 1.0
    hlim = jnp.float32(image_height) - 1.0
    supp, boxes_s = _run_nms(idx, deltas, wlim, hlim, jnp.float32(Wf))

    final_scores_all = jnp.where(supp, -1e9, scores)
    top_scores, top_idx = jax.lax.top_k(final_scores_all, _POST_NMS)
    final_boxes = boxes_s[top_idx]
    return final_boxes, top_scores
